# baseline (speedup 1.0000x reference)
import jax
import jax.numpy as jnp
from jax import lax
from jax.experimental import pallas as pl
from jax.experimental.pallas import tpu as pltpu
from jax.experimental.pallas import tpu_sc as plsc

B = 4
S = 2048
DIM = 1024
N = B * S
NC = 2
NS = 16
NW = NC * NS
SPW = S // NW
SCH = 8
NCHUNKS = SPW // SCH
GCH = B * SCH
ROWS_PER_W = B * SPW
LANES = 16
NBUF = 3


def _body(x_ref, tok_ref, pos_ref, out_ref,
          idx_v, tok_b, pos_b, sem_g, sem_p, sem_o):
    wid = lax.axis_index("s") * NC + lax.axis_index("c")
    sbase = wid * SPW

    for b in range(B):
        pltpu.sync_copy(x_ref.at[pl.ds(b * S + sbase, SPW)],
                        idx_v.at[pl.ds(b * SPW, SPW)])

    def start_in(c):
        i = c % NBUF
        for b in range(B):
            pltpu.async_copy(
                tok_ref.at[idx_v.at[pl.ds(b * SPW + c * SCH, SCH)]],
                tok_b.at[i, pl.ds(b * SCH, SCH)], sem_g)
        pltpu.async_copy(pos_ref.at[pl.ds(sbase + c * SCH, SCH)],
                         pos_b.at[i], sem_p)

    def wait_in(c):
        i = c % NBUF
        for b in range(B):
            pltpu.make_async_copy(
                tok_ref.at[idx_v.at[pl.ds(0, SCH)]],
                tok_b.at[i, pl.ds(b * SCH, SCH)], sem_g).wait()
        pltpu.make_async_copy(pos_ref.at[pl.ds(0, SCH)], pos_b.at[i],
                              sem_p).wait()

    def start_out(c):
        i = c % NBUF
        for b in range(B):
            pltpu.async_copy(
                tok_b.at[i, pl.ds(b * SCH, SCH)],
                out_ref.at[pl.ds(b * S + sbase + c * SCH, SCH)], sem_o)

    def drain_out(c):
        i = c % NBUF
        for b in range(B):
            pltpu.make_async_copy(
                tok_b.at[i, pl.ds(b * SCH, SCH)],
                out_ref.at[pl.ds(0, SCH)], sem_o).wait()

    def compute(c):
        i = c % NBUF

        @pl.loop(0, SCH)
        def _row(r):
            for j in range(DIM // LANES):
                sl = pl.ds(j * LANES, LANES)
                pv = pos_b[i, r, sl]
                for b in range(B):
                    tok_b[i, b * SCH + r, sl] = tok_b[i, b * SCH + r, sl] + pv

    start_in(0)

    @pl.loop(0, NCHUNKS)
    def _c(c):
        @pl.when(c >= 2)
        def _():
            drain_out(c - 2)

        @pl.when(c + 1 < NCHUNKS)
        def _():
            start_in(c + 1)

        wait_in(c)
        compute(c)
        start_out(c)

    drain_out(NCHUNKS - 2)
    drain_out(NCHUNKS - 1)


@jax.jit
def _run(xf, token_table, pos_table):
    mesh = plsc.VectorSubcoreMesh(core_axis_name="c", subcore_axis_name="s")
    return pl.kernel(
        _body,
        out_type=jax.ShapeDtypeStruct((N, DIM), jnp.float32),
        mesh=mesh,
        scratch_types=[
            pltpu.VMEM((ROWS_PER_W,), jnp.int32),
            pltpu.VMEM((NBUF, GCH, DIM), jnp.float32),
            pltpu.VMEM((NBUF, SCH, DIM), jnp.float32),
            pltpu.SemaphoreType.DMA,
            pltpu.SemaphoreType.DMA,
            pltpu.SemaphoreType.DMA,
        ],
    )(xf, token_table, pos_table)


def kernel(x, token_table, pos_table):
    xf = x.reshape(N).astype(jnp.int32)
    out = _run(xf, token_table, pos_table)
    return out.reshape(B, S, DIM)
